# baseline (speedup 1.0000x reference)
import functools

import jax
import jax.numpy as jnp
import numpy as np
from jax import lax
from jax.experimental import pallas as pl
from jax.experimental.pallas import tpu as pltpu
from jax.experimental.pallas import tpu_sc as plsc

H = 8
C = 16
NPAD = 10240
EPAD = 327680
NW = 32
EPW = EPAD // NW
CHUNK = 128
NCH = EPW // CHUNK
RPS = NPAD // 16
CH2 = 64
NCH2 = EPW // CH2

_mesh = plsc.VectorSubcoreMesh(
    core_axis_name="c", subcore_axis_name="s", num_cores=2, num_subcores=16)

_IN_BOUNDS = lax.GatherScatterMode.PROMISE_IN_BOUNDS


_DNUMS = lax.GatherDimensionNumbers(
    offset_dims=(), collapsed_slice_dims=(0,), start_index_map=(0,))


def _lane(vec, i):
    idx = jnp.full((16, 1), i, jnp.int32)
    return lax.gather(vec, idx, _DNUMS, (1,), mode=_IN_BOUNDS)


def _sc_pass1_body(first_layer, alS, alD, pack, ea, consts,
                   ex_out, denp,
                   pk_all, ea_sl, als0, als1, als2, als3, als4, als5, als6,
                   als7, ald0, ald1, ald2, ald3, ald4, ald5, ald6, ald7,
                   ex0, ex1, ex2, ex3, ex4, ex5, ex6, ex7,
                   consts_v, zero_v, den_sh, semG, semE, semD):
    cid = lax.axis_index("c")
    sid = lax.axis_index("s")
    wid = cid * 16 + sid
    als_b = (als0, als1, als2, als3, als4, als5, als6, als7)
    ald_b = (ald0, ald1, ald2, ald3, ald4, ald5, ald6, ald7)
    ex_b = (ex0, ex1, ex2, ex3, ex4, ex5, ex6, ex7)

    pltpu.sync_copy(pack.at[pl.ds(wid * 2 * NCH, 2 * NCH)], pk_all)
    pltpu.sync_copy(ea.at[pl.ds(wid * EPW, EPW)], ea_sl)

    z = jnp.zeros((16,), jnp.float32)

    def zbody(j, _):
        zero_v[j] = z
        return 0

    lax.fori_loop(0, RPS, zbody, 0, unroll=8)
    pltpu.sync_copy(zero_v, den_sh.at[pl.ds(sid * RPS, RPS)])
    pltpu.sync_copy(consts, consts_v)

    we = consts_v[0]
    maskA = consts_v[1]
    maskB = consts_v[2]
    maskC = consts_v[3]

    def compute(g, b):
        als_v = als_b[b]
        ald_v = ald_b[b]
        ex_v = ex_b[b]

        def grp_body(j16, _):
            eag = ea_sl[pl.ds(g * CHUNK + j16 * 16, 16)]
            for j in range(16):
                e = j16 * 16 + j
                eab = _lane(eag, j)
                alpha = als_v[e] + ald_v[e] + eab * we
                alpha = jnp.maximum(alpha, alpha * 0.2)
                ex = jnp.exp(alpha)
                if first_layer:
                    exs = ex * maskA + eab * maskB + maskC
                else:
                    exs = ex * maskA
                ex_v[e] = exs
            return 0

        lax.fori_loop(0, CHUNK // 16, grp_body, 0)

    plsc.subcore_barrier()

    K = 8

    def body(i, _):
        g0 = i * K
        gcps = []
        for b in range(K):
            g = g0 + b
            gcps.append((
                pltpu.async_copy(alS.at[pk_all.at[2 * g]], als_b[b], semG),
                pltpu.async_copy(alD.at[pk_all.at[2 * g + 1]], ald_b[b], semG),
            ))
        scps = []
        for b in range(K):
            g = g0 + b
            for cp in gcps[b]:
                cp.wait()
            compute(g, b)
            base = wid * EPW + g * CHUNK
            scps.append(pltpu.async_copy(
                ex_b[b], ex_out.at[pl.ds(base, CHUNK)], semE))
            scps.append(pltpu.async_copy(
                ex_b[b], den_sh.at[pk_all.at[2 * g + 1]], semD, add=True))
        for cp in scps:
            cp.wait()
        return 0

    lax.fori_loop(0, NCH // K, body, 0)

    plsc.subcore_barrier()
    pltpu.sync_copy(den_sh.at[pl.ds(sid * RPS, RPS)],
                    denp.at[cid, pl.ds(sid * RPS, RPS)])


def _make_sc_pass1(first_layer):
    return pl.kernel(
        functools.partial(_sc_pass1_body, first_layer),
        out_type=(
            jax.ShapeDtypeStruct((EPAD, 16), jnp.float32),
            jax.ShapeDtypeStruct((2, NPAD, 16), jnp.float32),
        ),
        mesh=_mesh,
        compiler_params=pltpu.CompilerParams(use_tc_tiling_on_sc=False),
        scratch_types=(
            [pltpu.VMEM((2 * NCH, CHUNK), jnp.int32),
             pltpu.VMEM((EPW,), jnp.float32)]
            + [pltpu.VMEM((CHUNK, 16), jnp.float32) for _ in range(24)]
            + [pltpu.VMEM((8, 16), jnp.float32),
               pltpu.VMEM((RPS, 16), jnp.float32),
               pltpu.VMEM_SHARED((NPAD, 16), jnp.float32),
               pltpu.SemaphoreType.DMA,
               pltpu.SemaphoreType.DMA,
               pltpu.SemaphoreType.DMA]
        ),
    )


_sc_pass1_l1 = _make_sc_pass1(True)
_sc_pass1_l23 = _make_sc_pass1(False)


def _sc_pass2_body(h, ex, pack,
                   outp,
                   pk_all, rows0, rows1, rows2, ex0, ex1, ex2,
                   out_sh, semG, semE, semS):
    cid = lax.axis_index("c")
    sid = lax.axis_index("s")
    wid = cid * 16 + sid
    rows_b = (rows0, rows1, rows2)
    ex_b = (ex0, ex1, ex2)

    pltpu.sync_copy(pack.at[pl.ds(wid * 2 * NCH2, 2 * NCH2)], pk_all)

    z = jnp.zeros((16,), jnp.float32)

    def zbody(t, _):
        for k in range(8):
            rows0[t, pl.ds(k * 16, 16)] = z
        return 0

    lax.fori_loop(0, CH2, zbody, 0, unroll=2)

    def zcopy(j, _):
        pltpu.sync_copy(rows0, out_sh.at[pl.ds(sid * RPS + j * CH2, CH2)])
        return 0

    lax.fori_loop(0, RPS // CH2, zcopy, 0)

    def idxrows(c):
        g = c // 2
        hh = c % 2
        return 4 * g + hh, 4 * g + 2 + hh

    def compute(c, b):
        rows_v = rows_b[b]
        ex_v = ex_b[b]

        def edge_body(e, _):
            att = ex_v[e]
            for hd in range(H):
                ab = _lane(att, hd)
                rows_v[e, pl.ds(hd * 16, 16)] = rows_v[e, pl.ds(hd * 16, 16)] * ab
            return 0

        lax.fori_loop(0, CH2, edge_body, 0, unroll=4)

    plsc.subcore_barrier()

    K = 3

    def body(i, _):
        c0 = i * K
        gcps = []
        for b in range(K):
            c = c0 + b
            base = wid * EPW + c * CH2
            rs, _ = idxrows(c)
            gcps.append((
                pltpu.async_copy(h.at[pk_all.at[rs]], rows_b[b], semG),
                pltpu.async_copy(ex.at[pl.ds(base, CH2)], ex_b[b], semE),
            ))
        scps = []
        for b in range(K):
            c = c0 + b
            for cp in gcps[b]:
                cp.wait()
            compute(c, b)
            _, rd = idxrows(c)
            scps.append(pltpu.async_copy(
                rows_b[b], out_sh.at[pk_all.at[rd]], semS, add=True))
        for cp in scps:
            cp.wait()
        return 0

    lax.fori_loop(0, NCH2 // K, body, 0)
    ct = NCH2 - NCH2 % K
    rs_t, rd_t = idxrows(ct)
    cpA = pltpu.async_copy(h.at[pk_all.at[rs_t]], rows_b[0], semG)
    cpB = pltpu.async_copy(ex.at[pl.ds(wid * EPW + ct * CH2, CH2)], ex_b[0],
                           semE)
    cpA.wait()
    cpB.wait()
    compute(ct, 0)
    pltpu.sync_copy(rows_b[0], out_sh.at[pk_all.at[rd_t]], add=True)

    plsc.subcore_barrier()

    def dump(j, _):
        r0 = sid * RPS + j * CH2
        pltpu.sync_copy(out_sh.at[pl.ds(r0, CH2)],
                        outp.at[cid, pl.ds(r0, CH2)])
        return 0

    lax.fori_loop(0, RPS // CH2, dump, 0)


_sc_pass2 = pl.kernel(
    _sc_pass2_body,
    out_type=jax.ShapeDtypeStruct((2, NPAD, 128), jnp.float32),
    mesh=_mesh,
    compiler_params=pltpu.CompilerParams(use_tc_tiling_on_sc=False),
    scratch_types=[
        pltpu.VMEM((2 * NCH2, CH2), jnp.int32),
        pltpu.VMEM((CH2, 128), jnp.float32),
        pltpu.VMEM((CH2, 128), jnp.float32),
        pltpu.VMEM((CH2, 128), jnp.float32),
        pltpu.VMEM((CH2, 16), jnp.float32),
        pltpu.VMEM((CH2, 16), jnp.float32),
        pltpu.VMEM((CH2, 16), jnp.float32),
        pltpu.VMEM_SHARED((NPAD, 128), jnp.float32),
        pltpu.SemaphoreType.DMA,
        pltpu.SemaphoreType.DMA,
        pltpu.SemaphoreType.DMA,
    ],
)


_BLK = 1024
_GRID = NPAD // _BLK


def _proj_body(x_ref, w_ref, as_ref, ad_ref, h_ref, als_ref, ald_ref):
    h = jnp.dot(x_ref[...], w_ref[...], preferred_element_type=jnp.float32)
    h_ref[...] = h
    als_ref[...] = jnp.dot(h, as_ref[...], preferred_element_type=jnp.float32)
    ald_ref[...] = jnp.dot(h, ad_ref[...], preferred_element_type=jnp.float32)


def _tc_proj(x, w, asm, adm):
    return pl.pallas_call(
        _proj_body,
        grid=(_GRID,),
        in_specs=[
            pl.BlockSpec((_BLK, 128), lambda i: (i, 0)),
            pl.BlockSpec((128, 128), lambda i: (0, 0)),
            pl.BlockSpec((128, 16), lambda i: (0, 0)),
            pl.BlockSpec((128, 16), lambda i: (0, 0)),
        ],
        out_specs=[
            pl.BlockSpec((_BLK, 128), lambda i: (i, 0)),
            pl.BlockSpec((_BLK, 16), lambda i: (i, 0)),
            pl.BlockSpec((_BLK, 16), lambda i: (i, 0)),
        ],
        out_shape=[
            jax.ShapeDtypeStruct((NPAD, 128), jnp.float32),
            jax.ShapeDtypeStruct((NPAD, 16), jnp.float32),
            jax.ShapeDtypeStruct((NPAD, 16), jnp.float32),
        ],
    )(x, w, asm, adm)


def _combine1_body(denp_ref, outp_ref, r_ref, b_ref, w_ref, as_ref, ad_ref,
                   we_ref, h_ref, als_ref, ald_ref, exself_ref, la_ref):
    den = denp_ref[0] + denp_ref[1]
    rden = 1.0 / (den + 1e-16)
    la = den[:, 9:10] / jnp.maximum(den[:, 8:9], 1.0)
    la_ref[...] = la
    rd128 = jnp.dot(rden, r_ref[...], preferred_element_type=jnp.float32)
    acc = outp_ref[0] + outp_ref[1]
    a = jnp.maximum(acc * rd128 + b_ref[...], 0.0)
    h = jnp.dot(a, w_ref[...], preferred_element_type=jnp.float32)
    h_ref[...] = h
    als = jnp.dot(h, as_ref[...], preferred_element_type=jnp.float32)
    ald = jnp.dot(h, ad_ref[...], preferred_element_type=jnp.float32)
    als_ref[...] = als
    ald_ref[...] = ald
    alpha = als + ald + la * we_ref[...]
    alpha = jnp.maximum(alpha, alpha * 0.2)
    lane = lax.broadcasted_iota(jnp.int32, (1, 16), 1)
    exself_ref[...] = jnp.where(lane < 8, jnp.exp(alpha), 0.0)


def _tc_combine1(denp, outp, rmat, b2d, w, asm, adm, we2d):
    return pl.pallas_call(
        _combine1_body,
        grid=(_GRID,),
        in_specs=[
            pl.BlockSpec((2, _BLK, 16), lambda i: (0, i, 0)),
            pl.BlockSpec((2, _BLK, 128), lambda i: (0, i, 0)),
            pl.BlockSpec((16, 128), lambda i: (0, 0)),
            pl.BlockSpec((1, 128), lambda i: (0, 0)),
            pl.BlockSpec((128, 128), lambda i: (0, 0)),
            pl.BlockSpec((128, 16), lambda i: (0, 0)),
            pl.BlockSpec((128, 16), lambda i: (0, 0)),
            pl.BlockSpec((1, 16), lambda i: (0, 0)),
        ],
        out_specs=[
            pl.BlockSpec((_BLK, 128), lambda i: (i, 0)),
            pl.BlockSpec((_BLK, 16), lambda i: (i, 0)),
            pl.BlockSpec((_BLK, 16), lambda i: (i, 0)),
            pl.BlockSpec((_BLK, 16), lambda i: (i, 0)),
            pl.BlockSpec((_BLK, 1), lambda i: (i, 0)),
        ],
        out_shape=[
            jax.ShapeDtypeStruct((NPAD, 128), jnp.float32),
            jax.ShapeDtypeStruct((NPAD, 16), jnp.float32),
            jax.ShapeDtypeStruct((NPAD, 16), jnp.float32),
            jax.ShapeDtypeStruct((NPAD, 16), jnp.float32),
            jax.ShapeDtypeStruct((NPAD, 1), jnp.float32),
        ],
    )(denp, outp, rmat, b2d, w, asm, adm, we2d)


def _combine2_body(denp_ref, outp_ref, exself_ref, hp_ref, la_ref, r_ref,
                   b_ref, w_ref, as_ref, ad_ref, we_ref,
                   h_ref, als_ref, ald_ref, exself2_ref):
    exs = exself_ref[...]
    den = denp_ref[0] + denp_ref[1] + exs
    rden = 1.0 / (den + 1e-16)
    rd128 = jnp.dot(rden, r_ref[...], preferred_element_type=jnp.float32)
    sm = hp_ref[...] * jnp.dot(exs, r_ref[...],
                               preferred_element_type=jnp.float32)
    acc = outp_ref[0] + outp_ref[1] + sm
    a = jnp.maximum(acc * rd128 + b_ref[...], 0.0)
    h = jnp.dot(a, w_ref[...], preferred_element_type=jnp.float32)
    h_ref[...] = h
    als = jnp.dot(h, as_ref[...], preferred_element_type=jnp.float32)
    ald = jnp.dot(h, ad_ref[...], preferred_element_type=jnp.float32)
    als_ref[...] = als
    ald_ref[...] = ald
    alpha = als + ald + la_ref[...] * we_ref[...]
    alpha = jnp.maximum(alpha, alpha * 0.2)
    lane = lax.broadcasted_iota(jnp.int32, (1, 16), 1)
    exself2_ref[...] = jnp.where(lane < 8, jnp.exp(alpha), 0.0)


def _tc_combine2(denp, outp, exself, hp, la, rmat, b2d, w, asm, adm, we2d):
    return pl.pallas_call(
        _combine2_body,
        grid=(_GRID,),
        in_specs=[
            pl.BlockSpec((2, _BLK, 16), lambda i: (0, i, 0)),
            pl.BlockSpec((2, _BLK, 128), lambda i: (0, i, 0)),
            pl.BlockSpec((_BLK, 16), lambda i: (i, 0)),
            pl.BlockSpec((_BLK, 128), lambda i: (i, 0)),
            pl.BlockSpec((_BLK, 1), lambda i: (i, 0)),
            pl.BlockSpec((16, 128), lambda i: (0, 0)),
            pl.BlockSpec((1, 128), lambda i: (0, 0)),
            pl.BlockSpec((128, 128), lambda i: (0, 0)),
            pl.BlockSpec((128, 16), lambda i: (0, 0)),
            pl.BlockSpec((128, 16), lambda i: (0, 0)),
            pl.BlockSpec((1, 16), lambda i: (0, 0)),
        ],
        out_specs=[
            pl.BlockSpec((_BLK, 128), lambda i: (i, 0)),
            pl.BlockSpec((_BLK, 16), lambda i: (i, 0)),
            pl.BlockSpec((_BLK, 16), lambda i: (i, 0)),
            pl.BlockSpec((_BLK, 16), lambda i: (i, 0)),
        ],
        out_shape=[
            jax.ShapeDtypeStruct((NPAD, 128), jnp.float32),
            jax.ShapeDtypeStruct((NPAD, 16), jnp.float32),
            jax.ShapeDtypeStruct((NPAD, 16), jnp.float32),
            jax.ShapeDtypeStruct((NPAD, 16), jnp.float32),
        ],
    )(denp, outp, exself, hp, la, rmat, b2d, w, asm, adm, we2d)


def _final_body(denp_ref, outp_ref, exself_ref, hp_ref, r_ref, b_ref, a_ref):
    exs = exself_ref[...]
    den = denp_ref[0] + denp_ref[1] + exs
    rden = 1.0 / (den + 1e-16)
    rd128 = jnp.dot(rden, r_ref[...], preferred_element_type=jnp.float32)
    sm = hp_ref[...] * jnp.dot(exs, r_ref[...],
                               preferred_element_type=jnp.float32)
    acc = outp_ref[0] + outp_ref[1] + sm
    a_ref[...] = jnp.maximum(acc * rd128 + b_ref[...], 0.0)


def _tc_final_act(denp, outp, exself, hp, rmat, b2d):
    return pl.pallas_call(
        _final_body,
        grid=(_GRID,),
        in_specs=[
            pl.BlockSpec((2, _BLK, 16), lambda i: (0, i, 0)),
            pl.BlockSpec((2, _BLK, 128), lambda i: (0, i, 0)),
            pl.BlockSpec((_BLK, 16), lambda i: (i, 0)),
            pl.BlockSpec((_BLK, 128), lambda i: (i, 0)),
            pl.BlockSpec((16, 128), lambda i: (0, 0)),
            pl.BlockSpec((1, 128), lambda i: (0, 0)),
        ],
        out_specs=pl.BlockSpec((_BLK, 128), lambda i: (i, 0)),
        out_shape=jax.ShapeDtypeStruct((NPAD, 128), jnp.float32),
    )(denp, outp, exself, hp, rmat, b2d)


def _pool_body(a_ref, gsum_ref):
    gsum_ref[0] = jnp.sum(a_ref[0], axis=0, keepdims=True) * (1.0 / 100.0)


def _tc_pool(a3g):
    return pl.pallas_call(
        _pool_body,
        grid=(100,),
        in_specs=[pl.BlockSpec((1, 100, 128), lambda g: (g, 0, 0))],
        out_specs=pl.BlockSpec((1, 1, 128), lambda g: (g, 0, 0)),
        out_shape=jax.ShapeDtypeStruct((100, 1, 128), jnp.float32),
    )(a3g)


def _fc_body(c_ref, w1_ref, b1_ref, w2_ref, b2_ref, o_ref):
    hid = jnp.dot(c_ref[...], w1_ref[...], preferred_element_type=jnp.float32)
    hid = jnp.maximum(hid + b1_ref[...], 0.0)
    o_ref[...] = jnp.dot(hid, w2_ref[...],
                         preferred_element_type=jnp.float32) + b2_ref[...]


def _tc_fc(comb, w1, b1, w2, b2):
    return pl.pallas_call(
        _fc_body,
        out_shape=jax.ShapeDtypeStruct((comb.shape[0], 8), jnp.float32),
    )(comb, w1, b1, w2, b2)


_BLKPAT = np.pad(np.repeat(np.eye(H, dtype=np.float32), C, axis=0),
                 ((0, 0), (0, 8)))
_RMAT = np.ascontiguousarray(_BLKPAT.T)


def _mk_as(a):
    return a.reshape(H * C, 1) * _BLKPAT


def _mk_we(We, ae):
    w = (We.reshape(H, C) * ae).sum(-1)
    return jnp.pad(w, (0, 8))


def kernel(x, edge_index, edge_attr, batch, num_graphs, actions,
           W1, as1, ad1, ae1, We1, b1,
           W2, as2, ad2, ae2, We2, b2,
           W3, as3, ad3, ae3, We3, b3,
           fc1W, fc1b, fc2W, fc2b):
    N = x.shape[0]
    E = edge_index.shape[1]
    G = actions.shape[0]

    padidx = N + (jnp.arange(EPAD - E, dtype=jnp.int32) % 8)
    src = jnp.concatenate([edge_index[0], padidx])
    dst = jnp.concatenate([edge_index[1], padidx])
    ea = jnp.pad(edge_attr[:, 0], (0, EPAD - E))
    pack = jnp.stack([src.reshape(-1, CHUNK),
                      dst.reshape(-1, CHUNK)], axis=1).reshape(-1, CHUNK)
    xp = jnp.pad(x, ((0, NPAD - N), (0, 0)))

    we1 = _mk_we(We1, ae1)
    we2 = _mk_we(We2, ae2)
    we3 = _mk_we(We3, ae3)
    maskA = jnp.concatenate([jnp.ones(8, jnp.float32), jnp.zeros(8, jnp.float32)])
    maskB = jnp.zeros(16, jnp.float32).at[9].set(1.0)
    maskC = jnp.zeros(16, jnp.float32).at[8].set(1.0)
    zrow = jnp.zeros(16, jnp.float32)

    def consts_for(we, first):
        rows = [we, maskA, maskB if first else zrow, maskC if first else zrow,
                zrow, zrow, zrow, zrow]
        return jnp.stack(rows)

    rmat = jnp.asarray(_RMAT)

    h1, alS1, alD1 = _tc_proj(xp, W1, _mk_as(as1), _mk_as(ad1))
    ex1, denp1 = _sc_pass1_l1(alS1, alD1, pack, ea, consts_for(we1, True))
    pack64 = pack.reshape(-1, CH2)
    outp1 = _sc_pass2(h1, ex1, pack64)

    h2, alS2, alD2, exself2, la = _tc_combine1(
        denp1, outp1, rmat, b1.reshape(1, 128), W2, _mk_as(as2),
        _mk_as(ad2), we2.reshape(1, 16))
    ex2, denp2 = _sc_pass1_l23(alS2, alD2, pack, ea, consts_for(we2, False))
    outp2 = _sc_pass2(h2, ex2, pack64)

    h3, alS3, alD3, exself3 = _tc_combine2(
        denp2, outp2, exself2, h2, la, rmat, b2.reshape(1, 128), W3,
        _mk_as(as3), _mk_as(ad3), we3.reshape(1, 16))
    ex3, denp3 = _sc_pass1_l23(alS3, alD3, pack, ea, consts_for(we3, False))
    outp3 = _sc_pass2(h3, ex3, pack64)

    a3 = _tc_final_act(denp3, outp3, exself3, h3, rmat, b3.reshape(1, 128))

    a3g = a3[:N].reshape(G, 100, 128)
    gsum = _tc_pool(a3g).reshape(G, 128)
    agent = a3g[:, :5, :].reshape(G, 640)
    comb = jnp.concatenate([agent, gsum, actions.reshape(G, -1)], axis=1)
    fc2Wp = jnp.pad(fc2W, ((0, 0), (0, 7)))
    fc2bp = jnp.pad(fc2b, (0, 7)).reshape(1, 8)
    out = _tc_fc(comb, fc1W, fc1b.reshape(1, 64), fc2Wp, fc2bp)
    return out[:, :1]


# speedup vs baseline: 1.6512x; 1.6512x over previous
import functools

import jax
import jax.numpy as jnp
import numpy as np
from jax import lax
from jax.experimental import pallas as pl
from jax.experimental.pallas import tpu as pltpu
from jax.experimental.pallas import tpu_sc as plsc

H = 8
C = 16
NPAD = 10240
EPAD = 327680
NW = 32
EPW = EPAD // NW
CHUNK = 128
NCH = EPW // CHUNK
RPS = NPAD // 16
CH2 = 64
NCH2 = EPW // CH2

_mesh = plsc.VectorSubcoreMesh(
    core_axis_name="c", subcore_axis_name="s", num_cores=2, num_subcores=16)

_IN_BOUNDS = lax.GatherScatterMode.PROMISE_IN_BOUNDS


_DNUMS = lax.GatherDimensionNumbers(
    offset_dims=(), collapsed_slice_dims=(0,), start_index_map=(0,))


def _lane(vec, i):
    idx = jnp.full((16, 1), i, jnp.int32)
    return lax.gather(vec, idx, _DNUMS, (1,), mode=_IN_BOUNDS)


def _sc_pass1_body(first_layer, alS, alD, pack, ea, consts,
                   ex_out, denp,
                   pk_all, ea_sl, als0, als1, als2, als3, als4, als5, als6,
                   als7, ald0, ald1, ald2, ald3, ald4, ald5, ald6, ald7,
                   ex0, ex1, ex2, ex3, ex4, ex5, ex6, ex7,
                   consts_v, zero_v, den_sh, semG, semE, semD):
    cid = lax.axis_index("c")
    sid = lax.axis_index("s")
    wid = cid * 16 + sid
    als_b = (als0, als1, als2, als3, als4, als5, als6, als7)
    ald_b = (ald0, ald1, ald2, ald3, ald4, ald5, ald6, ald7)
    ex_b = (ex0, ex1, ex2, ex3, ex4, ex5, ex6, ex7)

    pltpu.sync_copy(pack.at[pl.ds(wid * 2 * NCH, 2 * NCH)], pk_all)
    pltpu.sync_copy(ea.at[pl.ds(wid * EPW, EPW)], ea_sl)

    z = jnp.zeros((16,), jnp.float32)

    def zbody(j, _):
        zero_v[j] = z
        return 0

    lax.fori_loop(0, RPS, zbody, 0, unroll=8)
    pltpu.sync_copy(zero_v, den_sh.at[pl.ds(sid * RPS, RPS)])
    pltpu.sync_copy(consts, consts_v)

    we = consts_v[0]
    maskA = consts_v[1]
    maskB = consts_v[2]
    maskC = consts_v[3]

    def compute(g, b):
        als_v = als_b[b]
        ald_v = ald_b[b]
        ex_v = ex_b[b]

        def grp_body(j16, _):
            eag = ea_sl[pl.ds(g * CHUNK + j16 * 16, 16)]
            for j in range(16):
                e = j16 * 16 + j
                eab = _lane(eag, j)
                alpha = als_v[e] + ald_v[e] + eab * we
                alpha = jnp.maximum(alpha, alpha * 0.2)
                ex = jnp.exp(alpha)
                if first_layer:
                    exs = ex * maskA + eab * maskB + maskC
                else:
                    exs = ex * maskA
                ex_v[e] = exs
            return 0

        lax.fori_loop(0, CHUNK // 16, grp_body, 0)

    plsc.subcore_barrier()

    K = 8

    def body(i, _):
        g0 = i * K
        gcps = []
        for b in range(K):
            g = g0 + b
            gcps.append((
                pltpu.async_copy(alS.at[pk_all.at[2 * g]], als_b[b], semG),
                pltpu.async_copy(alD.at[pk_all.at[2 * g + 1]], ald_b[b], semG),
            ))
        scps = []
        for b in range(K):
            g = g0 + b
            for cp in gcps[b]:
                cp.wait()
            compute(g, b)
            base = wid * EPW + g * CHUNK
            scps.append(pltpu.async_copy(
                ex_b[b], ex_out.at[pl.ds(base, CHUNK)], semE))
            scps.append(pltpu.async_copy(
                ex_b[b], den_sh.at[pk_all.at[2 * g + 1]], semD, add=True))
        for cp in scps:
            cp.wait()
        return 0

    lax.fori_loop(0, NCH // K, body, 0)

    plsc.subcore_barrier()
    pltpu.sync_copy(den_sh.at[pl.ds(sid * RPS, RPS)],
                    denp.at[cid, pl.ds(sid * RPS, RPS)])


def _make_sc_pass1(first_layer):
    return pl.kernel(
        functools.partial(_sc_pass1_body, first_layer),
        out_type=(
            jax.ShapeDtypeStruct((EPAD, 16), jnp.float32),
            jax.ShapeDtypeStruct((2, NPAD, 16), jnp.float32),
        ),
        mesh=_mesh,
        compiler_params=pltpu.CompilerParams(use_tc_tiling_on_sc=False),
        scratch_types=(
            [pltpu.VMEM((2 * NCH, CHUNK), jnp.int32),
             pltpu.VMEM((EPW,), jnp.float32)]
            + [pltpu.VMEM((CHUNK, 16), jnp.float32) for _ in range(24)]
            + [pltpu.VMEM((8, 16), jnp.float32),
               pltpu.VMEM((RPS, 16), jnp.float32),
               pltpu.VMEM_SHARED((NPAD, 16), jnp.float32),
               pltpu.SemaphoreType.DMA,
               pltpu.SemaphoreType.DMA,
               pltpu.SemaphoreType.DMA]
        ),
    )


_sc_pass1_l1 = _make_sc_pass1(True)
_sc_pass1_l23 = _make_sc_pass1(False)


def _sc_pass2_body(h, ex, pack,
                   outp,
                   pk_all, rows0, rows1, rows2, ex0, ex1, ex2,
                   out_sh, semG, semE, semS):
    cid = lax.axis_index("c")
    sid = lax.axis_index("s")
    wid = cid * 16 + sid
    rows_b = (rows0, rows1, rows2)
    ex_b = (ex0, ex1, ex2)

    pltpu.sync_copy(pack.at[pl.ds(wid * 2 * NCH2, 2 * NCH2)], pk_all)

    z = jnp.zeros((16,), jnp.float32)

    def zbody(t, _):
        for k in range(8):
            rows0[t, pl.ds(k * 16, 16)] = z
        return 0

    lax.fori_loop(0, CH2, zbody, 0, unroll=2)

    def zcopy(j, _):
        pltpu.sync_copy(rows0, out_sh.at[pl.ds(sid * RPS + j * CH2, CH2)])
        return 0

    lax.fori_loop(0, RPS // CH2, zcopy, 0)

    def idxrows(c):
        g = c // 2
        hh = c % 2
        return 4 * g + hh, 4 * g + 2 + hh

    def compute(c, b):
        rows_v = rows_b[b]
        ex_v = ex_b[b]

        def edge_body(e, _):
            att = ex_v[e]
            for hd in range(H):
                ab = _lane(att, hd)
                rows_v[e, pl.ds(hd * 16, 16)] = rows_v[e, pl.ds(hd * 16, 16)] * ab
            return 0

        lax.fori_loop(0, CH2, edge_body, 0, unroll=2)

    plsc.subcore_barrier()

    K = 3

    def body(i, _):
        c0 = i * K
        gcps = []
        for b in range(K):
            c = c0 + b
            base = wid * EPW + c * CH2
            rs, _ = idxrows(c)
            gcps.append((
                pltpu.async_copy(h.at[pk_all.at[rs]], rows_b[b], semG),
                pltpu.async_copy(ex.at[pl.ds(base, CH2)], ex_b[b], semE),
            ))
        scps = []
        for b in range(K):
            c = c0 + b
            for cp in gcps[b]:
                cp.wait()
            compute(c, b)
            _, rd = idxrows(c)
            scps.append(pltpu.async_copy(
                rows_b[b], out_sh.at[pk_all.at[rd]], semS, add=True))
        for cp in scps:
            cp.wait()
        return 0

    lax.fori_loop(0, NCH2 // K, body, 0)
    ct = NCH2 - NCH2 % K
    rs_t, rd_t = idxrows(ct)
    cpA = pltpu.async_copy(h.at[pk_all.at[rs_t]], rows_b[0], semG)
    cpB = pltpu.async_copy(ex.at[pl.ds(wid * EPW + ct * CH2, CH2)], ex_b[0],
                           semE)
    cpA.wait()
    cpB.wait()
    compute(ct, 0)
    pltpu.sync_copy(rows_b[0], out_sh.at[pk_all.at[rd_t]], add=True)

    plsc.subcore_barrier()

    def dump(j, _):
        r0 = sid * RPS + j * CH2
        pltpu.sync_copy(out_sh.at[pl.ds(r0, CH2)],
                        outp.at[cid, pl.ds(r0, CH2)])
        return 0

    lax.fori_loop(0, RPS // CH2, dump, 0)


_sc_pass2 = pl.kernel(
    _sc_pass2_body,
    out_type=jax.ShapeDtypeStruct((2, NPAD, 128), jnp.float32),
    mesh=_mesh,
    compiler_params=pltpu.CompilerParams(use_tc_tiling_on_sc=False),
    scratch_types=[
        pltpu.VMEM((2 * NCH2, CH2), jnp.int32),
        pltpu.VMEM((CH2, 128), jnp.float32),
        pltpu.VMEM((CH2, 128), jnp.float32),
        pltpu.VMEM((CH2, 128), jnp.float32),
        pltpu.VMEM((CH2, 16), jnp.float32),
        pltpu.VMEM((CH2, 16), jnp.float32),
        pltpu.VMEM((CH2, 16), jnp.float32),
        pltpu.VMEM_SHARED((NPAD, 128), jnp.float32),
        pltpu.SemaphoreType.DMA,
        pltpu.SemaphoreType.DMA,
        pltpu.SemaphoreType.DMA,
    ],
)


_BLK = 1024
_GRID = NPAD // _BLK


def _proj_body(x_ref, w_ref, as_ref, ad_ref, h_ref, als_ref, ald_ref):
    h = jnp.dot(x_ref[...], w_ref[...], preferred_element_type=jnp.float32)
    h_ref[...] = h
    als_ref[...] = jnp.dot(h, as_ref[...], preferred_element_type=jnp.float32)
    ald_ref[...] = jnp.dot(h, ad_ref[...], preferred_element_type=jnp.float32)


def _tc_proj(x, w, asm, adm):
    return pl.pallas_call(
        _proj_body,
        grid=(_GRID,),
        in_specs=[
            pl.BlockSpec((_BLK, 128), lambda i: (i, 0)),
            pl.BlockSpec((128, 128), lambda i: (0, 0)),
            pl.BlockSpec((128, 16), lambda i: (0, 0)),
            pl.BlockSpec((128, 16), lambda i: (0, 0)),
        ],
        out_specs=[
            pl.BlockSpec((_BLK, 128), lambda i: (i, 0)),
            pl.BlockSpec((_BLK, 16), lambda i: (i, 0)),
            pl.BlockSpec((_BLK, 16), lambda i: (i, 0)),
        ],
        out_shape=[
            jax.ShapeDtypeStruct((NPAD, 128), jnp.float32),
            jax.ShapeDtypeStruct((NPAD, 16), jnp.float32),
            jax.ShapeDtypeStruct((NPAD, 16), jnp.float32),
        ],
    )(x, w, asm, adm)


def _combine1_body(denp_ref, outp_ref, r_ref, b_ref, w_ref, as_ref, ad_ref,
                   we_ref, h_ref, als_ref, ald_ref, exself_ref, la_ref):
    den = denp_ref[0] + denp_ref[1]
    rden = 1.0 / (den + 1e-16)
    la = den[:, 9:10] / jnp.maximum(den[:, 8:9], 1.0)
    la_ref[...] = la
    rd128 = jnp.dot(rden, r_ref[...], preferred_element_type=jnp.float32)
    acc = outp_ref[0] + outp_ref[1]
    a = jnp.maximum(acc * rd128 + b_ref[...], 0.0)
    h = jnp.dot(a, w_ref[...], preferred_element_type=jnp.float32)
    h_ref[...] = h
    als = jnp.dot(h, as_ref[...], preferred_element_type=jnp.float32)
    ald = jnp.dot(h, ad_ref[...], preferred_element_type=jnp.float32)
    als_ref[...] = als
    ald_ref[...] = ald
    alpha = als + ald + la * we_ref[...]
    alpha = jnp.maximum(alpha, alpha * 0.2)
    lane = lax.broadcasted_iota(jnp.int32, (1, 16), 1)
    exself_ref[...] = jnp.where(lane < 8, jnp.exp(alpha), 0.0)


def _tc_combine1(denp, outp, rmat, b2d, w, asm, adm, we2d):
    return pl.pallas_call(
        _combine1_body,
        grid=(_GRID,),
        in_specs=[
            pl.BlockSpec((2, _BLK, 16), lambda i: (0, i, 0)),
            pl.BlockSpec((2, _BLK, 128), lambda i: (0, i, 0)),
            pl.BlockSpec((16, 128), lambda i: (0, 0)),
            pl.BlockSpec((1, 128), lambda i: (0, 0)),
            pl.BlockSpec((128, 128), lambda i: (0, 0)),
            pl.BlockSpec((128, 16), lambda i: (0, 0)),
            pl.BlockSpec((128, 16), lambda i: (0, 0)),
            pl.BlockSpec((1, 16), lambda i: (0, 0)),
        ],
        out_specs=[
            pl.BlockSpec((_BLK, 128), lambda i: (i, 0)),
            pl.BlockSpec((_BLK, 16), lambda i: (i, 0)),
            pl.BlockSpec((_BLK, 16), lambda i: (i, 0)),
            pl.BlockSpec((_BLK, 16), lambda i: (i, 0)),
            pl.BlockSpec((_BLK, 1), lambda i: (i, 0)),
        ],
        out_shape=[
            jax.ShapeDtypeStruct((NPAD, 128), jnp.float32),
            jax.ShapeDtypeStruct((NPAD, 16), jnp.float32),
            jax.ShapeDtypeStruct((NPAD, 16), jnp.float32),
            jax.ShapeDtypeStruct((NPAD, 16), jnp.float32),
            jax.ShapeDtypeStruct((NPAD, 1), jnp.float32),
        ],
    )(denp, outp, rmat, b2d, w, asm, adm, we2d)


def _combine2_body(denp_ref, outp_ref, exself_ref, hp_ref, la_ref, r_ref,
                   b_ref, w_ref, as_ref, ad_ref, we_ref,
                   h_ref, als_ref, ald_ref, exself2_ref):
    exs = exself_ref[...]
    den = denp_ref[0] + denp_ref[1] + exs
    rden = 1.0 / (den + 1e-16)
    rd128 = jnp.dot(rden, r_ref[...], preferred_element_type=jnp.float32)
    sm = hp_ref[...] * jnp.dot(exs, r_ref[...],
                               preferred_element_type=jnp.float32)
    acc = outp_ref[0] + outp_ref[1] + sm
    a = jnp.maximum(acc * rd128 + b_ref[...], 0.0)
    h = jnp.dot(a, w_ref[...], preferred_element_type=jnp.float32)
    h_ref[...] = h
    als = jnp.dot(h, as_ref[...], preferred_element_type=jnp.float32)
    ald = jnp.dot(h, ad_ref[...], preferred_element_type=jnp.float32)
    als_ref[...] = als
    ald_ref[...] = ald
    alpha = als + ald + la_ref[...] * we_ref[...]
    alpha = jnp.maximum(alpha, alpha * 0.2)
    lane = lax.broadcasted_iota(jnp.int32, (1, 16), 1)
    exself2_ref[...] = jnp.where(lane < 8, jnp.exp(alpha), 0.0)


def _tc_combine2(denp, outp, exself, hp, la, rmat, b2d, w, asm, adm, we2d):
    return pl.pallas_call(
        _combine2_body,
        grid=(_GRID,),
        in_specs=[
            pl.BlockSpec((2, _BLK, 16), lambda i: (0, i, 0)),
            pl.BlockSpec((2, _BLK, 128), lambda i: (0, i, 0)),
            pl.BlockSpec((_BLK, 16), lambda i: (i, 0)),
            pl.BlockSpec((_BLK, 128), lambda i: (i, 0)),
            pl.BlockSpec((_BLK, 1), lambda i: (i, 0)),
            pl.BlockSpec((16, 128), lambda i: (0, 0)),
            pl.BlockSpec((1, 128), lambda i: (0, 0)),
            pl.BlockSpec((128, 128), lambda i: (0, 0)),
            pl.BlockSpec((128, 16), lambda i: (0, 0)),
            pl.BlockSpec((128, 16), lambda i: (0, 0)),
            pl.BlockSpec((1, 16), lambda i: (0, 0)),
        ],
        out_specs=[
            pl.BlockSpec((_BLK, 128), lambda i: (i, 0)),
            pl.BlockSpec((_BLK, 16), lambda i: (i, 0)),
            pl.BlockSpec((_BLK, 16), lambda i: (i, 0)),
            pl.BlockSpec((_BLK, 16), lambda i: (i, 0)),
        ],
        out_shape=[
            jax.ShapeDtypeStruct((NPAD, 128), jnp.float32),
            jax.ShapeDtypeStruct((NPAD, 16), jnp.float32),
            jax.ShapeDtypeStruct((NPAD, 16), jnp.float32),
            jax.ShapeDtypeStruct((NPAD, 16), jnp.float32),
        ],
    )(denp, outp, exself, hp, la, rmat, b2d, w, asm, adm, we2d)


def _final_body(denp_ref, outp_ref, exself_ref, hp_ref, r_ref, b_ref, a_ref):
    exs = exself_ref[...]
    den = denp_ref[0] + denp_ref[1] + exs
    rden = 1.0 / (den + 1e-16)
    rd128 = jnp.dot(rden, r_ref[...], preferred_element_type=jnp.float32)
    sm = hp_ref[...] * jnp.dot(exs, r_ref[...],
                               preferred_element_type=jnp.float32)
    acc = outp_ref[0] + outp_ref[1] + sm
    a_ref[...] = jnp.maximum(acc * rd128 + b_ref[...], 0.0)


def _tc_final_act(denp, outp, exself, hp, rmat, b2d):
    return pl.pallas_call(
        _final_body,
        grid=(_GRID,),
        in_specs=[
            pl.BlockSpec((2, _BLK, 16), lambda i: (0, i, 0)),
            pl.BlockSpec((2, _BLK, 128), lambda i: (0, i, 0)),
            pl.BlockSpec((_BLK, 16), lambda i: (i, 0)),
            pl.BlockSpec((_BLK, 128), lambda i: (i, 0)),
            pl.BlockSpec((16, 128), lambda i: (0, 0)),
            pl.BlockSpec((1, 128), lambda i: (0, 0)),
        ],
        out_specs=pl.BlockSpec((_BLK, 128), lambda i: (i, 0)),
        out_shape=jax.ShapeDtypeStruct((NPAD, 128), jnp.float32),
    )(denp, outp, exself, hp, rmat, b2d)


def _pool_body(a_ref, gsum_ref):
    gsum_ref[0] = jnp.sum(a_ref[0], axis=0, keepdims=True) * (1.0 / 100.0)


def _tc_pool(a3g):
    return pl.pallas_call(
        _pool_body,
        grid=(100,),
        in_specs=[pl.BlockSpec((1, 100, 128), lambda g: (g, 0, 0))],
        out_specs=pl.BlockSpec((1, 1, 128), lambda g: (g, 0, 0)),
        out_shape=jax.ShapeDtypeStruct((100, 1, 128), jnp.float32),
    )(a3g)


def _fc_body(c_ref, w1_ref, b1_ref, w2_ref, b2_ref, o_ref):
    hid = jnp.dot(c_ref[...], w1_ref[...], preferred_element_type=jnp.float32)
    hid = jnp.maximum(hid + b1_ref[...], 0.0)
    o_ref[...] = jnp.dot(hid, w2_ref[...],
                         preferred_element_type=jnp.float32) + b2_ref[...]


def _tc_fc(comb, w1, b1, w2, b2):
    return pl.pallas_call(
        _fc_body,
        out_shape=jax.ShapeDtypeStruct((comb.shape[0], 8), jnp.float32),
    )(comb, w1, b1, w2, b2)


_BLKPAT = np.pad(np.repeat(np.eye(H, dtype=np.float32), C, axis=0),
                 ((0, 0), (0, 8)))
_RMAT = np.ascontiguousarray(_BLKPAT.T)


def _mk_as(a):
    return a.reshape(H * C, 1) * _BLKPAT


def _mk_we(We, ae):
    w = (We.reshape(H, C) * ae).sum(-1)
    return jnp.pad(w, (0, 8))


def kernel(x, edge_index, edge_attr, batch, num_graphs, actions,
           W1, as1, ad1, ae1, We1, b1,
           W2, as2, ad2, ae2, We2, b2,
           W3, as3, ad3, ae3, We3, b3,
           fc1W, fc1b, fc2W, fc2b):
    N = x.shape[0]
    E = edge_index.shape[1]
    G = actions.shape[0]

    padidx = N + (jnp.arange(EPAD - E, dtype=jnp.int32) % 8)
    src = jnp.concatenate([edge_index[0], padidx])
    dst = jnp.concatenate([edge_index[1], padidx])
    ea = jnp.pad(edge_attr[:, 0], (0, EPAD - E))
    pack = jnp.stack([src.reshape(-1, CHUNK),
                      dst.reshape(-1, CHUNK)], axis=1).reshape(-1, CHUNK)
    xp = jnp.pad(x, ((0, NPAD - N), (0, 0)))

    we1 = _mk_we(We1, ae1)
    we2 = _mk_we(We2, ae2)
    we3 = _mk_we(We3, ae3)
    maskA = jnp.concatenate([jnp.ones(8, jnp.float32), jnp.zeros(8, jnp.float32)])
    maskB = jnp.zeros(16, jnp.float32).at[9].set(1.0)
    maskC = jnp.zeros(16, jnp.float32).at[8].set(1.0)
    zrow = jnp.zeros(16, jnp.float32)

    def consts_for(we, first):
        rows = [we, maskA, maskB if first else zrow, maskC if first else zrow,
                zrow, zrow, zrow, zrow]
        return jnp.stack(rows)

    rmat = jnp.asarray(_RMAT)

    h1, alS1, alD1 = _tc_proj(xp, W1, _mk_as(as1), _mk_as(ad1))
    ex1, denp1 = _sc_pass1_l1(alS1, alD1, pack, ea, consts_for(we1, True))
    pack64 = pack.reshape(-1, CH2)
    outp1 = _sc_pass2(h1, ex1, pack64)

    h2, alS2, alD2, exself2, la = _tc_combine1(
        denp1, outp1, rmat, b1.reshape(1, 128), W2, _mk_as(as2),
        _mk_as(ad2), we2.reshape(1, 16))
    ex2, denp2 = _sc_pass1_l23(alS2, alD2, pack, ea, consts_for(we2, False))
    outp2 = _sc_pass2(h2, ex2, pack64)

    h3, alS3, alD3, exself3 = _tc_combine2(
        denp2, outp2, exself2, h2, la, rmat, b2.reshape(1, 128), W3,
        _mk_as(as3), _mk_as(ad3), we3.reshape(1, 16))
    ex3, denp3 = _sc_pass1_l23(alS3, alD3, pack, ea, consts_for(we3, False))
    outp3 = _sc_pass2(h3, ex3, pack64)

    a3 = _tc_final_act(denp3, outp3, exself3, h3, rmat, b3.reshape(1, 128))

    a3g = a3[:N].reshape(G, 100, 128)
    gsum = _tc_pool(a3g).reshape(G, 128)
    agent = a3g[:, :5, :].reshape(G, 640)
    comb = jnp.concatenate([agent, gsum, actions.reshape(G, -1)], axis=1)
    fc2Wp = jnp.pad(fc2W, ((0, 0), (0, 7)))
    fc2bp = jnp.pad(fc2b, (0, 7)).reshape(1, 8)
    out = _tc_fc(comb, fc1W, fc1b.reshape(1, 64), fc2Wp, fc2bp)
    return out[:, :1]


# speedup vs baseline: 1.6739x; 1.0138x over previous
import functools

import jax
import jax.numpy as jnp
import numpy as np
from jax import lax
from jax.experimental import pallas as pl
from jax.experimental.pallas import tpu as pltpu
from jax.experimental.pallas import tpu_sc as plsc

H = 8
C = 16
NPAD = 10240
EPAD = 327680
NW = 32
EPW = EPAD // NW
CHUNK = 128
NCH = EPW // CHUNK
RPS = NPAD // 16
CH2 = 64
NCH2 = EPW // CH2

_mesh = plsc.VectorSubcoreMesh(
    core_axis_name="c", subcore_axis_name="s", num_cores=2, num_subcores=16)

_IN_BOUNDS = lax.GatherScatterMode.PROMISE_IN_BOUNDS


_DNUMS = lax.GatherDimensionNumbers(
    offset_dims=(), collapsed_slice_dims=(0,), start_index_map=(0,))


def _lane(vec, i):
    idx = jnp.full((16, 1), i, jnp.int32)
    return lax.gather(vec, idx, _DNUMS, (1,), mode=_IN_BOUNDS)


def _sc_pass1_body(first_layer, alS, alD, pack, ea, consts,
                   ex_out, denp,
                   pk_all, ea_sl, als0, als1, als2, als3, als4, als5, als6,
                   als7, ald0, ald1, ald2, ald3, ald4, ald5, ald6, ald7,
                   ex0, ex1, ex2, ex3, ex4, ex5, ex6, ex7,
                   consts_v, zero_v, den_sh, semG, semE, semD):
    cid = lax.axis_index("c")
    sid = lax.axis_index("s")
    wid = cid * 16 + sid
    als_b = (als0, als1, als2, als3, als4, als5, als6, als7)
    ald_b = (ald0, ald1, ald2, ald3, ald4, ald5, ald6, ald7)
    ex_b = (ex0, ex1, ex2, ex3, ex4, ex5, ex6, ex7)

    pltpu.sync_copy(pack.at[pl.ds(wid * 2 * NCH, 2 * NCH)], pk_all)
    pltpu.sync_copy(ea.at[pl.ds(wid * EPW, EPW)], ea_sl)

    z = jnp.zeros((16,), jnp.float32)

    def zbody(j, _):
        zero_v[j] = z
        return 0

    lax.fori_loop(0, RPS, zbody, 0, unroll=8)
    pltpu.sync_copy(zero_v, den_sh.at[pl.ds(sid * RPS, RPS)])
    pltpu.sync_copy(consts, consts_v)

    we = consts_v[0]
    maskA = consts_v[1]
    maskB = consts_v[2]
    maskC = consts_v[3]

    def compute(g, b):
        als_v = als_b[b]
        ald_v = ald_b[b]
        ex_v = ex_b[b]

        def grp_body(j16, _):
            eag = ea_sl[pl.ds(g * CHUNK + j16 * 16, 16)]
            for j in range(16):
                e = j16 * 16 + j
                eab = _lane(eag, j)
                alpha = als_v[e] + ald_v[e] + eab * we
                alpha = jnp.maximum(alpha, alpha * 0.2)
                ex = jnp.exp(alpha)
                if first_layer:
                    exs = ex * maskA + eab * maskB + maskC
                else:
                    exs = ex * maskA
                ex_v[e] = exs
            return 0

        lax.fori_loop(0, CHUNK // 16, grp_body, 0)

    plsc.subcore_barrier()

    K = 8

    def body(i, _):
        g0 = i * K
        gcps = []
        for b in range(K):
            g = g0 + b
            gcps.append((
                pltpu.async_copy(alS.at[pk_all.at[2 * g]], als_b[b], semG),
                pltpu.async_copy(alD.at[pk_all.at[2 * g + 1]], ald_b[b], semG),
            ))
        scps = []
        for b in range(K):
            g = g0 + b
            for cp in gcps[b]:
                cp.wait()
            compute(g, b)
            base = wid * EPW + g * CHUNK
            scps.append(pltpu.async_copy(
                ex_b[b], ex_out.at[pl.ds(base, CHUNK)], semE))
            scps.append(pltpu.async_copy(
                ex_b[b], den_sh.at[pk_all.at[2 * g + 1]], semD, add=True))
        for cp in scps:
            cp.wait()
        return 0

    lax.fori_loop(0, NCH // K, body, 0)

    plsc.subcore_barrier()
    pltpu.sync_copy(den_sh.at[pl.ds(sid * RPS, RPS)],
                    denp.at[cid, pl.ds(sid * RPS, RPS)])


def _make_sc_pass1(first_layer):
    return pl.kernel(
        functools.partial(_sc_pass1_body, first_layer),
        out_type=(
            jax.ShapeDtypeStruct((EPAD, 16), jnp.float32),
            jax.ShapeDtypeStruct((2, NPAD, 16), jnp.float32),
        ),
        mesh=_mesh,
        compiler_params=pltpu.CompilerParams(use_tc_tiling_on_sc=False),
        scratch_types=(
            [pltpu.VMEM((2 * NCH, CHUNK), jnp.int32),
             pltpu.VMEM((EPW,), jnp.float32)]
            + [pltpu.VMEM((CHUNK, 16), jnp.float32) for _ in range(24)]
            + [pltpu.VMEM((8, 16), jnp.float32),
               pltpu.VMEM((RPS, 16), jnp.float32),
               pltpu.VMEM_SHARED((NPAD, 16), jnp.float32),
               pltpu.SemaphoreType.DMA,
               pltpu.SemaphoreType.DMA,
               pltpu.SemaphoreType.DMA]
        ),
    )


_sc_pass1_l1 = _make_sc_pass1(True)
_sc_pass1_l23 = _make_sc_pass1(False)


def _sc_pass2_body(h, ex, pack,
                   outp,
                   pk_all, rows0, rows1, rows2, ex0, ex1, ex2,
                   out_sh, semG, semE, semS):
    cid = lax.axis_index("c")
    sid = lax.axis_index("s")
    wid = cid * 16 + sid
    rows_b = (rows0, rows1, rows2)
    ex_b = (ex0, ex1, ex2)

    pltpu.sync_copy(pack.at[pl.ds(wid * 2 * NCH2, 2 * NCH2)], pk_all)

    z = jnp.zeros((16,), jnp.float32)

    def zbody(t, _):
        for k in range(8):
            rows0[t, pl.ds(k * 16, 16)] = z
        return 0

    lax.fori_loop(0, CH2, zbody, 0, unroll=2)

    def zcopy(j, _):
        pltpu.sync_copy(rows0, out_sh.at[pl.ds(sid * RPS + j * CH2, CH2)])
        return 0

    lax.fori_loop(0, RPS // CH2, zcopy, 0)

    def idxrows(c):
        g = c // 2
        hh = c % 2
        return 4 * g + hh, 4 * g + 2 + hh

    def compute(c, b):
        rows_v = rows_b[b]
        ex_v = ex_b[b]

        def edge_body(e, _):
            att = ex_v[e]
            for hd in range(H):
                ab = _lane(att, hd)
                rows_v[e, pl.ds(hd * 16, 16)] = rows_v[e, pl.ds(hd * 16, 16)] * ab
            return 0

        lax.fori_loop(0, CH2, edge_body, 0, unroll=2)

    plsc.subcore_barrier()

    K = 3

    def body(i, _):
        c0 = i * K
        gcps = []
        for b in range(K):
            c = c0 + b
            base = wid * EPW + c * CH2
            rs, _ = idxrows(c)
            gcps.append((
                pltpu.async_copy(h.at[pk_all.at[rs]], rows_b[b], semG),
                pltpu.async_copy(ex.at[pl.ds(base, CH2)], ex_b[b], semE),
            ))
        scps = []
        for b in range(K):
            c = c0 + b
            for cp in gcps[b]:
                cp.wait()
            compute(c, b)
            _, rd = idxrows(c)
            scps.append(pltpu.async_copy(
                rows_b[b], out_sh.at[pk_all.at[rd]], semS, add=True))
        for cp in scps:
            cp.wait()
        return 0

    lax.fori_loop(0, NCH2 // K, body, 0)
    ct = NCH2 - NCH2 % K
    rs_t, rd_t = idxrows(ct)
    cpA = pltpu.async_copy(h.at[pk_all.at[rs_t]], rows_b[0], semG)
    cpB = pltpu.async_copy(ex.at[pl.ds(wid * EPW + ct * CH2, CH2)], ex_b[0],
                           semE)
    cpA.wait()
    cpB.wait()
    compute(ct, 0)
    pltpu.sync_copy(rows_b[0], out_sh.at[pk_all.at[rd_t]], add=True)

    plsc.subcore_barrier()

    def dump(j, _):
        r0 = sid * RPS + j * CH2
        pltpu.sync_copy(out_sh.at[pl.ds(r0, CH2)],
                        outp.at[cid, pl.ds(r0, CH2)])
        return 0

    lax.fori_loop(0, RPS // CH2, dump, 0)


_sc_pass2 = pl.kernel(
    _sc_pass2_body,
    out_type=jax.ShapeDtypeStruct((2, NPAD, 128), jnp.float32),
    mesh=_mesh,
    compiler_params=pltpu.CompilerParams(use_tc_tiling_on_sc=False),
    scratch_types=[
        pltpu.VMEM((2 * NCH2, CH2), jnp.int32),
        pltpu.VMEM((CH2, 128), jnp.float32),
        pltpu.VMEM((CH2, 128), jnp.float32),
        pltpu.VMEM((CH2, 128), jnp.float32),
        pltpu.VMEM((CH2, 16), jnp.float32),
        pltpu.VMEM((CH2, 16), jnp.float32),
        pltpu.VMEM((CH2, 16), jnp.float32),
        pltpu.VMEM_SHARED((NPAD, 128), jnp.float32),
        pltpu.SemaphoreType.DMA,
        pltpu.SemaphoreType.DMA,
        pltpu.SemaphoreType.DMA,
    ],
)


_BLK = 1024
_GRID = NPAD // _BLK


def _proj_body(x_ref, w_ref, as_ref, ad_ref, h_ref, als_ref, ald_ref):
    h = jnp.dot(x_ref[...], w_ref[...], preferred_element_type=jnp.float32)
    h_ref[...] = h
    als_ref[...] = jnp.dot(h, as_ref[...], preferred_element_type=jnp.float32)
    ald_ref[...] = jnp.dot(h, ad_ref[...], preferred_element_type=jnp.float32)


def _tc_proj(x, w, asm, adm):
    return pl.pallas_call(
        _proj_body,
        grid=(_GRID,),
        in_specs=[
            pl.BlockSpec((_BLK, 128), lambda i: (i, 0)),
            pl.BlockSpec((128, 128), lambda i: (0, 0)),
            pl.BlockSpec((128, 16), lambda i: (0, 0)),
            pl.BlockSpec((128, 16), lambda i: (0, 0)),
        ],
        out_specs=[
            pl.BlockSpec((_BLK, 128), lambda i: (i, 0)),
            pl.BlockSpec((_BLK, 16), lambda i: (i, 0)),
            pl.BlockSpec((_BLK, 16), lambda i: (i, 0)),
        ],
        out_shape=[
            jax.ShapeDtypeStruct((NPAD, 128), jnp.float32),
            jax.ShapeDtypeStruct((NPAD, 16), jnp.float32),
            jax.ShapeDtypeStruct((NPAD, 16), jnp.float32),
        ],
    )(x, w, asm, adm)


def _den1_body(denp_ref, rden_ref, la_ref):
    den = denp_ref[0] + denp_ref[1]
    rden_ref[...] = 1.0 / (den + 1e-16)
    la_ref[...] = den[:, 9:10] / jnp.maximum(den[:, 8:9], 1.0)


def _tc_den1(denp):
    return pl.pallas_call(
        _den1_body,
        grid=(_GRID,),
        in_specs=[pl.BlockSpec((2, _BLK, 16), lambda i: (0, i, 0))],
        out_specs=[
            pl.BlockSpec((_BLK, 16), lambda i: (i, 0)),
            pl.BlockSpec((_BLK, 1), lambda i: (i, 0)),
        ],
        out_shape=[
            jax.ShapeDtypeStruct((NPAD, 16), jnp.float32),
            jax.ShapeDtypeStruct((NPAD, 1), jnp.float32),
        ],
    )(denp)


def _den23_body(denp_ref, exself_ref, h_ref, r_ref, rden_ref, sm_ref):
    den = denp_ref[0] + denp_ref[1] + exself_ref[...]
    rden = 1.0 / (den + 1e-16)
    rden_ref[...] = rden
    sm_ref[...] = h_ref[...] * jnp.dot(exself_ref[...], r_ref[...],
                                       preferred_element_type=jnp.float32)


def _tc_den23(denp, exself, h, rmat):
    return pl.pallas_call(
        _den23_body,
        grid=(_GRID,),
        in_specs=[
            pl.BlockSpec((2, _BLK, 16), lambda i: (0, i, 0)),
            pl.BlockSpec((_BLK, 16), lambda i: (i, 0)),
            pl.BlockSpec((_BLK, 128), lambda i: (i, 0)),
            pl.BlockSpec((16, 128), lambda i: (0, 0)),
        ],
        out_specs=[
            pl.BlockSpec((_BLK, 16), lambda i: (i, 0)),
            pl.BlockSpec((_BLK, 128), lambda i: (i, 0)),
        ],
        out_shape=[
            jax.ShapeDtypeStruct((NPAD, 16), jnp.float32),
            jax.ShapeDtypeStruct((NPAD, 128), jnp.float32),
        ],
    )(denp, exself, h, rmat)


def _combine_body(has_sm, outp_ref, sm_ref, rden_ref, r_ref, b_ref, w_ref,
                  as_ref, ad_ref, la_ref, we_ref,
                  h_ref, als_ref, ald_ref, exself_ref):
    acc = outp_ref[0] + outp_ref[1]
    if has_sm:
        acc = acc + sm_ref[...]
    rd128 = jnp.dot(rden_ref[...], r_ref[...],
                    preferred_element_type=jnp.float32)
    a = jnp.maximum(acc * rd128 + b_ref[...], 0.0)
    h = jnp.dot(a, w_ref[...], preferred_element_type=jnp.float32)
    h_ref[...] = h
    als = jnp.dot(h, as_ref[...], preferred_element_type=jnp.float32)
    ald = jnp.dot(h, ad_ref[...], preferred_element_type=jnp.float32)
    als_ref[...] = als
    ald_ref[...] = ald
    alpha = als + ald + la_ref[...] * we_ref[...]
    alpha = jnp.maximum(alpha, alpha * 0.2)
    lane = lax.broadcasted_iota(jnp.int32, (1, 16), 1)
    exself_ref[...] = jnp.where(lane < 8, jnp.exp(alpha), 0.0)


def _tc_combine(outp, sm, rden, rmat, b2d, w, asm, adm, la, we2d, has_sm):
    return pl.pallas_call(
        functools.partial(_combine_body, has_sm),
        grid=(_GRID,),
        in_specs=[
            pl.BlockSpec((2, _BLK, 128), lambda i: (0, i, 0)),
            pl.BlockSpec((_BLK, 128), lambda i: (i, 0)),
            pl.BlockSpec((_BLK, 16), lambda i: (i, 0)),
            pl.BlockSpec((16, 128), lambda i: (0, 0)),
            pl.BlockSpec((1, 128), lambda i: (0, 0)),
            pl.BlockSpec((128, 128), lambda i: (0, 0)),
            pl.BlockSpec((128, 16), lambda i: (0, 0)),
            pl.BlockSpec((128, 16), lambda i: (0, 0)),
            pl.BlockSpec((_BLK, 1), lambda i: (i, 0)),
            pl.BlockSpec((1, 16), lambda i: (0, 0)),
        ],
        out_specs=[
            pl.BlockSpec((_BLK, 128), lambda i: (i, 0)),
            pl.BlockSpec((_BLK, 16), lambda i: (i, 0)),
            pl.BlockSpec((_BLK, 16), lambda i: (i, 0)),
            pl.BlockSpec((_BLK, 16), lambda i: (i, 0)),
        ],
        out_shape=[
            jax.ShapeDtypeStruct((NPAD, 128), jnp.float32),
            jax.ShapeDtypeStruct((NPAD, 16), jnp.float32),
            jax.ShapeDtypeStruct((NPAD, 16), jnp.float32),
            jax.ShapeDtypeStruct((NPAD, 16), jnp.float32),
        ],
    )(outp, sm, rden, rmat, b2d, w, asm, adm, la, we2d)


def _final_body(outp_ref, sm_ref, rden_ref, r_ref, b_ref, a_ref):
    acc = outp_ref[0] + outp_ref[1] + sm_ref[...]
    rd128 = jnp.dot(rden_ref[...], r_ref[...],
                    preferred_element_type=jnp.float32)
    a_ref[...] = jnp.maximum(acc * rd128 + b_ref[...], 0.0)


def _tc_final_act(outp, sm, rden, rmat, b2d):
    return pl.pallas_call(
        _final_body,
        grid=(_GRID,),
        in_specs=[
            pl.BlockSpec((2, _BLK, 128), lambda i: (0, i, 0)),
            pl.BlockSpec((_BLK, 128), lambda i: (i, 0)),
            pl.BlockSpec((_BLK, 16), lambda i: (i, 0)),
            pl.BlockSpec((16, 128), lambda i: (0, 0)),
            pl.BlockSpec((1, 128), lambda i: (0, 0)),
        ],
        out_specs=pl.BlockSpec((_BLK, 128), lambda i: (i, 0)),
        out_shape=jax.ShapeDtypeStruct((NPAD, 128), jnp.float32),
    )(outp, sm, rden, rmat, b2d)


def _pool_body(a_ref, gsum_ref):
    gsum_ref[0] = jnp.sum(a_ref[0], axis=0, keepdims=True) * (1.0 / 100.0)


def _tc_pool(a3g):
    return pl.pallas_call(
        _pool_body,
        grid=(100,),
        in_specs=[pl.BlockSpec((1, 100, 128), lambda g: (g, 0, 0))],
        out_specs=pl.BlockSpec((1, 1, 128), lambda g: (g, 0, 0)),
        out_shape=jax.ShapeDtypeStruct((100, 1, 128), jnp.float32),
    )(a3g)


def _fc_body(c_ref, w1_ref, b1_ref, w2_ref, b2_ref, o_ref):
    hid = jnp.dot(c_ref[...], w1_ref[...], preferred_element_type=jnp.float32)
    hid = jnp.maximum(hid + b1_ref[...], 0.0)
    o_ref[...] = jnp.dot(hid, w2_ref[...],
                         preferred_element_type=jnp.float32) + b2_ref[...]


def _tc_fc(comb, w1, b1, w2, b2):
    return pl.pallas_call(
        _fc_body,
        out_shape=jax.ShapeDtypeStruct((comb.shape[0], 8), jnp.float32),
    )(comb, w1, b1, w2, b2)


_BLKPAT = np.pad(np.repeat(np.eye(H, dtype=np.float32), C, axis=0),
                 ((0, 0), (0, 8)))
_RMAT = np.ascontiguousarray(_BLKPAT.T)


def _mk_as(a):
    return a.reshape(H * C, 1) * _BLKPAT


def _mk_we(We, ae):
    w = (We.reshape(H, C) * ae).sum(-1)
    return jnp.pad(w, (0, 8))


def kernel(x, edge_index, edge_attr, batch, num_graphs, actions,
           W1, as1, ad1, ae1, We1, b1,
           W2, as2, ad2, ae2, We2, b2,
           W3, as3, ad3, ae3, We3, b3,
           fc1W, fc1b, fc2W, fc2b):
    N = x.shape[0]
    E = edge_index.shape[1]
    G = actions.shape[0]

    padidx = N + (jnp.arange(EPAD - E, dtype=jnp.int32) % 8)
    src = jnp.concatenate([edge_index[0], padidx])
    dst = jnp.concatenate([edge_index[1], padidx])
    ea = jnp.pad(edge_attr[:, 0], (0, EPAD - E))
    pack = jnp.stack([src.reshape(-1, CHUNK),
                      dst.reshape(-1, CHUNK)], axis=1).reshape(-1, CHUNK)
    xp = jnp.pad(x, ((0, NPAD - N), (0, 0)))

    we1 = _mk_we(We1, ae1)
    we2 = _mk_we(We2, ae2)
    we3 = _mk_we(We3, ae3)
    maskA = jnp.concatenate([jnp.ones(8, jnp.float32), jnp.zeros(8, jnp.float32)])
    maskB = jnp.zeros(16, jnp.float32).at[9].set(1.0)
    maskC = jnp.zeros(16, jnp.float32).at[8].set(1.0)
    zrow = jnp.zeros(16, jnp.float32)

    def consts_for(we, first):
        rows = [we, maskA, maskB if first else zrow, maskC if first else zrow,
                zrow, zrow, zrow, zrow]
        return jnp.stack(rows)

    rmat = jnp.asarray(_RMAT)

    h1, alS1, alD1 = _tc_proj(xp, W1, _mk_as(as1), _mk_as(ad1))
    ex1, denp1 = _sc_pass1_l1(alS1, alD1, pack, ea, consts_for(we1, True))
    pack64 = pack.reshape(-1, CH2)
    rden1, la = _tc_den1(denp1)
    outp1 = _sc_pass2(h1, ex1, pack64)

    zsm = jnp.zeros((NPAD, 128), jnp.float32)
    h2, alS2, alD2, exself2 = _tc_combine(
        outp1, zsm, rden1, rmat, b1.reshape(1, 128), W2, _mk_as(as2),
        _mk_as(ad2), la, we2.reshape(1, 16), has_sm=False)
    ex2, denp2 = _sc_pass1_l23(alS2, alD2, pack, ea, consts_for(we2, False))
    rden2, sm2 = _tc_den23(denp2, exself2, h2, rmat)
    outp2 = _sc_pass2(h2, ex2, pack64)

    h3, alS3, alD3, exself3 = _tc_combine(
        outp2, sm2, rden2, rmat, b2.reshape(1, 128), W3, _mk_as(as3),
        _mk_as(ad3), la, we3.reshape(1, 16), has_sm=True)
    ex3, denp3 = _sc_pass1_l23(alS3, alD3, pack, ea, consts_for(we3, False))
    rden3, sm3 = _tc_den23(denp3, exself3, h3, rmat)
    outp3 = _sc_pass2(h3, ex3, pack64)

    a3 = _tc_final_act(outp3, sm3, rden3, rmat, b3.reshape(1, 128))

    a3g = a3[:N].reshape(G, 100, 128)
    gsum = _tc_pool(a3g).reshape(G, 128)
    agent = a3g[:, :5, :].reshape(G, 640)
    comb = jnp.concatenate([agent, gsum, actions.reshape(G, -1)], axis=1)
    fc2Wp = jnp.pad(fc2W, ((0, 0), (0, 7)))
    fc2bp = jnp.pad(fc2b, (0, 7)).reshape(1, 8)
    out = _tc_fc(comb, fc1W, fc1b.reshape(1, 64), fc2Wp, fc2bp)
    return out[:, :1]
